# baseline (speedup 1.0000x reference)
import functools

import jax
import jax.numpy as jnp
from jax import lax
from jax.experimental import pallas as pl
from jax.experimental.pallas import tpu as pltpu
from jax.experimental.pallas import tpu_sc as plsc

_NUM_EMBEDDINGS = 100000
_SEQ_LEN = 200
_EMB_DIM = 64
_BATCH = 4096

_NW = 32
_CH = 100
_CH_PAD = 104
_BATCH_PER_W = _BATCH // _NW
_CHUNKS_PER_W = 2 * _BATCH_PER_W


def _position_embedding_host():
    even_index = jnp.arange(0, _EMB_DIM, 2, dtype=jnp.float32)
    denominator = jnp.power(10000.0, even_index / _EMB_DIM)
    positions = jnp.arange(0, _SEQ_LEN, dtype=jnp.float32).reshape(_SEQ_LEN, 1)
    even_pe = jnp.sin(positions / denominator)
    odd_pe = jnp.cos(positions / denominator)
    stacked = jnp.stack([even_pe, odd_pe], axis=2)
    return stacked.reshape(_SEQ_LEN, _EMB_DIM)


_TCOLS = _NUM_EMBEDDINGS // 128
_TAIL0 = _TCOLS * 128
_TAIL = _NUM_EMBEDDINGS - _TAIL0


def _relayout_body(tabT_hbm, tail_hbm, out_hbm, stg, obuf, tail_v, dsem):
    nc = 2
    wid = lax.axis_index("s") * nc + lax.axis_index("c")
    start = wid * 24 + jnp.minimum(wid, 13)
    count = 24 + (wid < 13).astype(jnp.int32)
    ii = lax.iota(jnp.int32, 16)

    def col(g, carry):
        goff = pl.multiple_of(g * 128, 128)
        for k in range(8):
            pltpu.async_copy(
                tabT_hbm.at[pl.ds(8 * k, 8), pl.ds(goff, 128)],
                stg.at[pl.ds(8 * k, 8)], dsem)
        for k in range(8):
            pltpu.make_async_copy(
                tabT_hbm.at[pl.ds(0, 8), pl.ds(0, 128)],
                stg.at[pl.ds(0, 8)], dsem).wait()

        def row(i, c2):
            for c in range(_EMB_DIM // 16):
                vals = plsc.load_gather(stg, [16 * c + ii, jnp.full((16,), i, jnp.int32)])
                obuf[i, pl.ds(16 * c, 16)] = vals
            return c2

        lax.fori_loop(0, 128, row, 0, unroll=2)
        pltpu.sync_copy(obuf, out_hbm.at[pl.ds(goff, 128)])
        return carry

    lax.fori_loop(start, start + count, col, 0)

    @pl.when(wid == _NW - 1)
    def _():
        pltpu.sync_copy(tail_hbm, tail_v)

        def trow(i, c2):
            for c in range(_EMB_DIM // 16):
                sl = pl.ds(16 * c, 16)
                obuf[i, sl] = tail_v[i, sl]
            return c2

        lax.fori_loop(0, _TAIL, trow, 0)
        pltpu.sync_copy(obuf.at[pl.ds(0, _TAIL)],
                        out_hbm.at[pl.ds(_TAIL0, _TAIL)])


def _sc_body(table_hbm, idx_hbm, pos_hbm, out_hbm,
             idx_v, pos_sh, buf_a, buf_b, sem_a, sem_b, psem_a, psem_b):
    nc = 2
    sid = lax.axis_index("s")
    wid = sid * nc + lax.axis_index("c")
    chunk0 = wid * _CHUNKS_PER_W
    batch0 = wid * _BATCH_PER_W
    last_even = _CHUNKS_PER_W - 2

    @pl.when(sid == 0)
    def _():
        pltpu.sync_copy(pos_hbm, pos_sh)

    pltpu.sync_copy(idx_hbm.at[pl.ds(chunk0, _CHUNKS_PER_W)], idx_v)
    plsc.subcore_barrier()

    def prefill(buf, psem, half):
        pltpu.async_copy(pos_sh.at[pl.ds(half * _CH, _CH)], buf, psem)

    def fire(g, buf, sem, psem, half):
        pltpu.make_async_copy(
            pos_sh.at[pl.ds(half * _CH, _CH)], buf, psem).wait()
        return pltpu.async_copy(table_hbm.at[idx_v.at[g]], buf, sem, add=True)

    prefill(buf_a, psem_a, 0)
    prefill(buf_b, psem_b, 1)
    fire(0, buf_a, sem_a, psem_a, 0)

    def body(go, carry):
        g = 2 * go
        b = batch0 + go
        fire(g + 1, buf_b, sem_b, psem_b, 1)
        pltpu.make_async_copy(table_hbm.at[idx_v.at[0]], buf_a, sem_a).wait()
        pltpu.sync_copy(buf_a, out_hbm.at[b, pl.ds(0, _CH)])
        prefill(buf_a, psem_a, 0)
        fire(jnp.minimum(g + 2, last_even), buf_a, sem_a, psem_a, 0)
        pltpu.make_async_copy(table_hbm.at[idx_v.at[0]], buf_b, sem_b).wait()
        pltpu.sync_copy(buf_b, out_hbm.at[b, pl.ds(_CH, _CH)])
        prefill(buf_b, psem_b, 1)
        return carry

    lax.fori_loop(0, _BATCH_PER_W, body, 0)
    pltpu.make_async_copy(table_hbm.at[idx_v.at[0]], buf_a, sem_a).wait()
    pltpu.make_async_copy(pos_sh.at[pl.ds(_CH, _CH)], buf_b, psem_b).wait()


@jax.jit
def kernel(input, table):
    pos = _position_embedding_host()
    idx2d = (input * 2).reshape(_BATCH * 2, _CH)

    mesh = plsc.VectorSubcoreMesh(core_axis_name="c", subcore_axis_name="s")
    table_p = pl.kernel(
        _relayout_body,
        out_type=jax.ShapeDtypeStruct((_NUM_EMBEDDINGS, 128), jnp.float32),
        mesh=mesh,
        scratch_types=[
            pltpu.VMEM((_EMB_DIM, 128), jnp.float32),
            pltpu.VMEM((128, 128), jnp.float32),
            pltpu.VMEM((_TAIL, _EMB_DIM), jnp.float32),
            pltpu.SemaphoreType.DMA,
        ],
        compiler_params=pltpu.CompilerParams(
            use_tc_tiling_on_sc=True, needs_layout_passes=False
        ),
    )(table.T, lax.slice(table, (_TAIL0, 0), (_NUM_EMBEDDINGS, _EMB_DIM)))
    table = table_p.reshape(2 * _NUM_EMBEDDINGS, _EMB_DIM)
    out = pl.kernel(
        _sc_body,
        out_type=jax.ShapeDtypeStruct((_BATCH, _SEQ_LEN, _EMB_DIM), jnp.float32),
        mesh=mesh,
        scratch_types=[
            pltpu.VMEM((_CHUNKS_PER_W, _CH), jnp.int32),
            pltpu.VMEM_SHARED((_SEQ_LEN, _EMB_DIM), jnp.float32),
            pltpu.VMEM((_CH, _EMB_DIM), jnp.float32),
            pltpu.VMEM((_CH, _EMB_DIM), jnp.float32),
            pltpu.SemaphoreType.DMA,
            pltpu.SemaphoreType.DMA,
            pltpu.SemaphoreType.DMA,
            pltpu.SemaphoreType.DMA,
        ],
        compiler_params=pltpu.CompilerParams(use_tc_tiling_on_sc=False),
    )(table, idx2d, pos)
    return out


# speedup vs baseline: 1.1825x; 1.1825x over previous
import functools

import jax
import jax.numpy as jnp
from jax import lax
from jax.experimental import layout as jlayout
from jax.experimental import pallas as pl
from jax.experimental.pallas import tpu as pltpu
from jax.experimental.pallas import tpu_sc as plsc

_NUM_EMBEDDINGS = 100000
_SEQ_LEN = 200
_EMB_DIM = 64
_BATCH = 4096

_NW = 32
_CH = 100
_CH_PAD = 104
_BATCH_PER_W = _BATCH // _NW
_CHUNKS_PER_W = 2 * _BATCH_PER_W


def _position_embedding_host():
    even_index = jnp.arange(0, _EMB_DIM, 2, dtype=jnp.float32)
    denominator = jnp.power(10000.0, even_index / _EMB_DIM)
    positions = jnp.arange(0, _SEQ_LEN, dtype=jnp.float32).reshape(_SEQ_LEN, 1)
    even_pe = jnp.sin(positions / denominator)
    odd_pe = jnp.cos(positions / denominator)
    stacked = jnp.stack([even_pe, odd_pe], axis=2)
    return stacked.reshape(_SEQ_LEN, _EMB_DIM)


def _sc_body(table_hbm, idx_hbm, pos_hbm, out_hbm,
             idx_v, pos_sh, buf_a, buf_b, sem_a, sem_b, psem_a, psem_b):
    nc = 2
    sid = lax.axis_index("s")
    wid = sid * nc + lax.axis_index("c")
    chunk0 = wid * _CHUNKS_PER_W
    batch0 = wid * _BATCH_PER_W
    last_even = _CHUNKS_PER_W - 2

    @pl.when(sid == 0)
    def _():
        pltpu.sync_copy(pos_hbm, pos_sh)

    pltpu.sync_copy(idx_hbm.at[pl.ds(chunk0, _CHUNKS_PER_W)], idx_v)
    plsc.subcore_barrier()

    def prefill(buf, psem, half):
        pltpu.async_copy(pos_sh.at[pl.ds(half * _CH, _CH)], buf, psem)

    def fire(g, buf, sem, psem, half):
        pltpu.make_async_copy(
            pos_sh.at[pl.ds(half * _CH, _CH)], buf, psem).wait()
        return pltpu.async_copy(table_hbm.at[idx_v.at[g]], buf, sem, add=True)

    prefill(buf_a, psem_a, 0)
    prefill(buf_b, psem_b, 1)
    fire(0, buf_a, sem_a, psem_a, 0)

    def body(go, carry):
        g = 2 * go
        b = batch0 + go
        fire(g + 1, buf_b, sem_b, psem_b, 1)
        pltpu.make_async_copy(table_hbm.at[idx_v.at[0]], buf_a, sem_a).wait()
        pltpu.sync_copy(buf_a, out_hbm.at[b, pl.ds(0, _CH)])
        prefill(buf_a, psem_a, 0)
        fire(jnp.minimum(g + 2, last_even), buf_a, sem_a, psem_a, 0)
        pltpu.make_async_copy(table_hbm.at[idx_v.at[0]], buf_b, sem_b).wait()
        pltpu.sync_copy(buf_b, out_hbm.at[b, pl.ds(_CH, _CH)])
        prefill(buf_b, psem_b, 1)
        return carry

    lax.fori_loop(0, _BATCH_PER_W, body, 0)
    pltpu.make_async_copy(table_hbm.at[idx_v.at[0]], buf_a, sem_a).wait()
    pltpu.make_async_copy(pos_sh.at[pl.ds(_CH, _CH)], buf_b, psem_b).wait()


def _impl(input, table):
    pos = _position_embedding_host()
    idx2d = (input * 2).reshape(_BATCH * 2, _CH)

    mesh = plsc.VectorSubcoreMesh(core_axis_name="c", subcore_axis_name="s")
    table = jnp.pad(table, ((0, 0), (0, 128 - _EMB_DIM))).reshape(
        2 * _NUM_EMBEDDINGS, _EMB_DIM
    )
    out = pl.kernel(
        _sc_body,
        out_type=jax.ShapeDtypeStruct((_BATCH, _SEQ_LEN, _EMB_DIM), jnp.float32),
        mesh=mesh,
        scratch_types=[
            pltpu.VMEM((_CHUNKS_PER_W, _CH), jnp.int32),
            pltpu.VMEM_SHARED((_SEQ_LEN, _EMB_DIM), jnp.float32),
            pltpu.VMEM((_CH, _EMB_DIM), jnp.float32),
            pltpu.VMEM((_CH, _EMB_DIM), jnp.float32),
            pltpu.SemaphoreType.DMA,
            pltpu.SemaphoreType.DMA,
            pltpu.SemaphoreType.DMA,
            pltpu.SemaphoreType.DMA,
        ],
        compiler_params=pltpu.CompilerParams(use_tc_tiling_on_sc=False),
    )(table, idx2d, pos)
    return out


_jit_cache = {}


def kernel(input, table):
    f = _jit_cache.get("f")
    if f is None:
        fmt = jlayout.Format(
            jlayout.Layout(major_to_minor=(0, 1, 2), tiling=()),
            jax.sharding.SingleDeviceSharding(jax.devices()[0]),
        )
        f = jax.jit(_impl, out_shardings=fmt)
        _jit_cache["f"] = f
    return f(input, table)
